# Initial kernel scaffold; baseline (speedup 1.0000x reference)
import functools

import jax
import jax.numpy as jnp
from jax import lax
from jax.experimental import pallas as pl
from jax.experimental.pallas import tpu as pltpu
from jax.experimental.pallas import tpu_sc as plsc

N = 10000
E = 320000
D = 128

NUM_CORES = 2
NUM_SUBCORES = 16
NUM_WORKERS = NUM_CORES * NUM_SUBCORES
EDGES_PER_WORKER = E // NUM_WORKERS
CHUNK = 80
CHUNKS_PER_WORKER = EDGES_PER_WORKER // CHUNK
ROWS_PER_TILE = N // NUM_SUBCORES
LANES = 16


def _sc_propagate(x, src, dst, w, zeros):
    mesh = plsc.VectorSubcoreMesh(core_axis_name="c", subcore_axis_name="s")

    @functools.partial(
        pl.kernel,
        out_type=jax.ShapeDtypeStruct((NUM_CORES * N, D), jnp.float32),
        mesh=mesh,
        scratch_types=[
            pltpu.VMEM((CHUNK,), jnp.int32),
            pltpu.VMEM((CHUNK,), jnp.int32),
            pltpu.VMEM((CHUNK,), jnp.float32),
            pltpu.VMEM((CHUNK, D), jnp.float32),
            pltpu.VMEM_SHARED((N, D), jnp.float32),
        ],
    )
    def k(x_hbm, src_hbm, dst_hbm, w_hbm, z_hbm, out_hbm,
          idx_s, idx_d, wts, rows, acc):
        cid = lax.axis_index("c")
        sid = lax.axis_index("s")
        wid = cid * NUM_SUBCORES + sid

        row0 = sid * ROWS_PER_TILE
        pltpu.sync_copy(z_hbm.at[pl.ds(row0, ROWS_PER_TILE)],
                        acc.at[pl.ds(row0, ROWS_PER_TILE)])
        plsc.subcore_barrier()

        @pl.loop(0, CHUNKS_PER_WORKER)
        def _(kk):
            base = wid * EDGES_PER_WORKER + kk * CHUNK
            pltpu.sync_copy(src_hbm.at[pl.ds(base, CHUNK)], idx_s)
            pltpu.sync_copy(dst_hbm.at[pl.ds(base, CHUNK)], idx_d)
            pltpu.sync_copy(w_hbm.at[pl.ds(base, CHUNK)], wts)
            pltpu.sync_copy(x_hbm.at[idx_s], rows)
            for e in range(CHUNK):
                wb = plsc.load_gather(
                    wts, [jnp.full((LANES,), e, jnp.int32)])
                for j in range(D // LANES):
                    sl = pl.ds(j * LANES, LANES)
                    rows[e, sl] = rows[e, sl] * wb
            pltpu.sync_copy(rows, acc.at[idx_d], add=True)

        plsc.subcore_barrier()
        out_base = cid * N + sid * ROWS_PER_TILE
        pltpu.sync_copy(acc.at[pl.ds(row0, ROWS_PER_TILE)],
                        out_hbm.at[pl.ds(out_base, ROWS_PER_TILE)])

    return k(x, src, dst, w, zeros)


def _tc_finish_body(p0_ref, p1_ref, w_ref, o_ref):
    s = p0_ref[...] + p1_ref[...]
    o_ref[...] = jnp.dot(s, w_ref[...], preferred_element_type=jnp.float32)


def _tc_finish(partials, W):
    blk = 1000
    grid = (N // blk,)
    return pl.pallas_call(
        _tc_finish_body,
        grid=grid,
        in_specs=[
            pl.BlockSpec((blk, D), lambda i: (i, 0)),
            pl.BlockSpec((blk, D), lambda i: (i + N // blk, 0)),
            pl.BlockSpec((D, D), lambda i: (0, 0)),
        ],
        out_specs=pl.BlockSpec((blk, D), lambda i: (i, 0)),
        out_shape=jax.ShapeDtypeStruct((N, D), jnp.float32),
    )(partials, partials, W)


def kernel(x, edge_index, edge_weight, W):
    dst = edge_index[0]
    src = edge_index[1]
    zeros = jnp.zeros((N, D), jnp.float32)
    partials = _sc_propagate(x, src, dst, edge_weight, zeros)
    return _tc_finish(partials, W)


# speedup vs baseline: 4.0771x; 4.0771x over previous
import dataclasses
import functools

import jax
import jax.numpy as jnp
from jax import lax
from jax.experimental import pallas as pl
from jax.experimental.pallas import tpu as pltpu
from jax.experimental.pallas import tpu_sc as plsc

N = 10000
E = 320000
D = 128

NUM_CORES = 2
NUM_SUBCORES = 16
NUM_WORKERS = NUM_CORES * NUM_SUBCORES
EDGES_PER_WORKER = E // NUM_WORKERS
CHUNK = 80
CHUNKS_PER_WORKER = EDGES_PER_WORKER // CHUNK
ROWS_PER_TILE = 624
TAIL_ROW0 = ROWS_PER_TILE * NUM_SUBCORES
TAIL_ROWS = N - TAIL_ROW0
LANES = 16


def _sc_propagate(x, src, dst, w, zeros):
    mesh = plsc.VectorSubcoreMesh(core_axis_name="c", subcore_axis_name="s")
    cp = pltpu.CompilerParams()
    if "needs_layout_passes" in pltpu.CompilerParams.__dataclass_fields__:
        cp = dataclasses.replace(cp, needs_layout_passes=False)

    @functools.partial(
        pl.kernel,
        compiler_params=cp,
        out_type=jax.ShapeDtypeStruct((NUM_CORES * N, D), jnp.float32),
        mesh=mesh,
        scratch_types=[
            pltpu.VMEM((CHUNK,), jnp.int32),
            pltpu.VMEM((CHUNK,), jnp.int32),
            pltpu.VMEM((CHUNK + LANES,), jnp.float32),
            pltpu.VMEM((CHUNK, D), jnp.float32),
            pltpu.VMEM_SHARED((N, D), jnp.float32),
        ],
    )
    def k(x_hbm, src_hbm, dst_hbm, w_hbm, z_hbm, out_hbm,
          idx_s, idx_d, wts, rows, acc):
        cid = lax.axis_index("c")
        sid = lax.axis_index("s")
        wid = cid * NUM_SUBCORES + sid

        row0 = sid * ROWS_PER_TILE
        pltpu.sync_copy(z_hbm.at[pl.ds(row0, ROWS_PER_TILE)],
                        acc.at[pl.ds(row0, ROWS_PER_TILE)])

        @pl.when(sid == NUM_SUBCORES - 1)
        def _():
            pltpu.sync_copy(z_hbm.at[pl.ds(TAIL_ROW0, TAIL_ROWS)],
                            acc.at[pl.ds(TAIL_ROW0, TAIL_ROWS)])

        plsc.subcore_barrier()

        @pl.loop(0, CHUNKS_PER_WORKER)
        def _(kk):
            base = wid * EDGES_PER_WORKER + kk * CHUNK
            pltpu.sync_copy(src_hbm.at[pl.ds(base, CHUNK)], idx_s)
            pltpu.sync_copy(dst_hbm.at[pl.ds(base, CHUNK)], idx_d)
            pltpu.sync_copy(w_hbm.at[pl.ds(base, CHUNK)],
                            wts.at[pl.ds(LANES, CHUNK)])
            pltpu.sync_copy(x_hbm.at[idx_s], rows)
            for e in range(CHUNK):
                wb = plsc.load_gather(
                    wts, [jnp.full((LANES,), LANES + e, jnp.int32)])
                for j in range(D // LANES):
                    sl = pl.ds(j * LANES, LANES)
                    rows[e, sl] = rows[e, sl] * wb
            pltpu.sync_copy(rows, acc.at[idx_d], add=True)

        plsc.subcore_barrier()
        out_base = cid * N + sid * ROWS_PER_TILE
        pltpu.sync_copy(acc.at[pl.ds(row0, ROWS_PER_TILE)],
                        out_hbm.at[pl.ds(out_base, ROWS_PER_TILE)])

        @pl.when(sid == NUM_SUBCORES - 1)
        def _():
            pltpu.sync_copy(acc.at[pl.ds(TAIL_ROW0, TAIL_ROWS)],
                            out_hbm.at[pl.ds(cid * N + TAIL_ROW0, TAIL_ROWS)])

    return k(x, src, dst, w, zeros)


def _tc_finish_body(p0_ref, p1_ref, w_ref, o_ref):
    s = p0_ref[...] + p1_ref[...]
    o_ref[...] = jnp.dot(s, w_ref[...], preferred_element_type=jnp.float32)


def _tc_finish(partials, W):
    blk = 1000
    grid = (N // blk,)
    return pl.pallas_call(
        _tc_finish_body,
        grid=grid,
        in_specs=[
            pl.BlockSpec((blk, D), lambda i: (i, 0)),
            pl.BlockSpec((blk, D), lambda i: (i + N // blk, 0)),
            pl.BlockSpec((D, D), lambda i: (0, 0)),
        ],
        out_specs=pl.BlockSpec((blk, D), lambda i: (i, 0)),
        out_shape=jax.ShapeDtypeStruct((N, D), jnp.float32),
    )(partials, partials, W)


def kernel(x, edge_index, edge_weight, W):
    dst = edge_index[0]
    src = edge_index[1]
    zeros = jnp.zeros((N, D), jnp.float32)
    partials = _sc_propagate(x, src, dst, edge_weight, zeros)
    return _tc_finish(partials, W)


# speedup vs baseline: 4.1766x; 1.0244x over previous
import dataclasses
import functools

import jax
import jax.numpy as jnp
from jax import lax
from jax.experimental import pallas as pl
from jax.experimental.pallas import tpu as pltpu
from jax.experimental.pallas import tpu_sc as plsc

N = 10000
E = 320000
D = 128

NUM_CORES = 2
NUM_SUBCORES = 16
NUM_WORKERS = NUM_CORES * NUM_SUBCORES
EDGES_PER_WORKER = E // NUM_WORKERS
CHUNK = 80
K = EDGES_PER_WORKER // CHUNK
KALL = E // CHUNK
ROWS_PER_TILE = 624
TAIL_ROW0 = ROWS_PER_TILE * NUM_SUBCORES
TAIL_ROWS = N - TAIL_ROW0
LANES = 16
SW = 2 * CHUNK


def _sc_propagate(x, srcw, dst, zeros):
    mesh = plsc.VectorSubcoreMesh(core_axis_name="c", subcore_axis_name="s")
    cp = pltpu.CompilerParams()
    if "needs_layout_passes" in pltpu.CompilerParams.__dataclass_fields__:
        cp = dataclasses.replace(cp, needs_layout_passes=False)

    @functools.partial(
        pl.kernel,
        compiler_params=cp,
        out_type=jax.ShapeDtypeStruct((NUM_CORES * N, D), jnp.float32),
        mesh=mesh,
        scratch_types=(
            [pltpu.VMEM((SW,), jnp.int32) for _ in range(4)]
            + [pltpu.VMEM((CHUNK,), jnp.int32) for _ in range(4)]
            + [pltpu.VMEM((CHUNK, D), jnp.float32) for _ in range(2)]
            + [pltpu.VMEM_SHARED((N, D), jnp.float32)]
            + [pltpu.SemaphoreType.DMA for _ in range(8)]
        ),
    )
    def k(x_hbm, sw_hbm, dst_hbm, z_hbm, out_hbm,
          s0, s1, s2, s3, d0, d1, d2, d3, r0, r1, acc,
          se0, se1, se2, se3, sg0, sg1, ss0, ss1):
        sbuf = [s0, s1, s2, s3]
        dbuf = [d0, d1, d2, d3]
        rows = [r0, r1]
        se = [se0, se1, se2, se3]
        sg = [sg0, sg1]
        ss = [ss0, ss1]

        cid = lax.axis_index("c")
        sid = lax.axis_index("s")
        wid = cid * NUM_SUBCORES + sid

        def emit_e(j, eb):
            gw = wid * K + j
            pltpu.async_copy(sw_hbm.at[pl.ds(gw * SW, SW)], sbuf[eb], se[eb])
            pltpu.async_copy(dst_hbm.at[pl.ds(wid * EDGES_PER_WORKER + j * CHUNK,
                                              CHUNK)], dbuf[eb], se[eb])

        def wait_e(eb):
            pltpu.make_async_copy(sw_hbm.at[pl.ds(0, SW)], sbuf[eb],
                                  se[eb]).wait()
            pltpu.make_async_copy(dst_hbm.at[pl.ds(0, CHUNK)], dbuf[eb],
                                  se[eb]).wait()

        def emit_g(eb, rb):
            pltpu.async_copy(x_hbm.at[sbuf[eb].at[pl.ds(0, CHUNK)]],
                             rows[rb], sg[rb])

        def wait_g(rb):
            pltpu.make_async_copy(x_hbm.at[pl.ds(0, CHUNK)], rows[rb],
                                  sg[rb]).wait()

        def emit_s(eb, rb):
            pltpu.async_copy(rows[rb], acc.at[dbuf[eb]], ss[rb], add=True)

        def wait_s(rb):
            pltpu.make_async_copy(rows[rb], acc.at[pl.ds(0, CHUNK)],
                                  ss[rb]).wait()

        def compute(eb, rb):
            for e in range(CHUNK):
                wb_i = plsc.load_gather(
                    sbuf[eb], [jnp.full((LANES,), CHUNK + e, jnp.int32)])
                wb = plsc.bitcast(wb_i, jnp.float32)
                for j in range(D // LANES):
                    sl = pl.ds(j * LANES, LANES)
                    rows[rb][e, sl] = rows[rb][e, sl] * wb

        row0 = sid * ROWS_PER_TILE
        pltpu.sync_copy(z_hbm.at[pl.ds(row0, ROWS_PER_TILE)],
                        acc.at[pl.ds(row0, ROWS_PER_TILE)])

        @pl.when(sid == NUM_SUBCORES - 1)
        def _():
            pltpu.sync_copy(z_hbm.at[pl.ds(TAIL_ROW0, TAIL_ROWS)],
                            acc.at[pl.ds(TAIL_ROW0, TAIL_ROWS)])

        emit_e(0, 0)
        emit_e(1, 1)
        plsc.subcore_barrier()
        wait_e(0)
        emit_g(0, 0)

        @pl.loop(0, K - 1, step=4)
        def _(c0):
            for b in range(4):
                j = c0 + b
                rb = b % 2
                ob = 1 - rb
                wait_g(rb)
                compute(b, rb)
                emit_s(b, rb)

                @pl.when(j + 2 < K)
                def _():
                    emit_e(j + 2, (b + 2) % 4)

                if b == 0:
                    @pl.when(j >= 1)
                    def _():
                        wait_s(ob)
                else:
                    wait_s(ob)
                wait_e((b + 1) % 4)
                emit_g((b + 1) % 4, ob)

        wait_g(0)
        compute(0, 0)
        emit_s(0, 0)
        wait_s(1)
        wait_s(0)

        plsc.subcore_barrier()
        out_base = cid * N + sid * ROWS_PER_TILE
        pltpu.sync_copy(acc.at[pl.ds(row0, ROWS_PER_TILE)],
                        out_hbm.at[pl.ds(out_base, ROWS_PER_TILE)])

        @pl.when(sid == NUM_SUBCORES - 1)
        def _():
            pltpu.sync_copy(acc.at[pl.ds(TAIL_ROW0, TAIL_ROWS)],
                            out_hbm.at[pl.ds(cid * N + TAIL_ROW0, TAIL_ROWS)])

    return k(x, srcw, dst, zeros)


def _tc_finish_body(p0_ref, p1_ref, w_ref, o_ref):
    s = p0_ref[...] + p1_ref[...]
    o_ref[...] = jnp.dot(s, w_ref[...], preferred_element_type=jnp.float32)


def _tc_finish(partials, W):
    blk = 1000
    grid = (N // blk,)
    return pl.pallas_call(
        _tc_finish_body,
        grid=grid,
        in_specs=[
            pl.BlockSpec((blk, D), lambda i: (i, 0)),
            pl.BlockSpec((blk, D), lambda i: (i + N // blk, 0)),
            pl.BlockSpec((D, D), lambda i: (0, 0)),
        ],
        out_specs=pl.BlockSpec((blk, D), lambda i: (i, 0)),
        out_shape=jax.ShapeDtypeStruct((N, D), jnp.float32),
    )(partials, partials, W)


def kernel(x, edge_index, edge_weight, W):
    dst = edge_index[0]
    src = edge_index[1]
    srcw = jnp.concatenate(
        [src.reshape(KALL, CHUNK),
         jax.lax.bitcast_convert_type(edge_weight, jnp.int32).reshape(
             KALL, CHUNK)],
        axis=1).reshape(KALL * SW)
    zeros = jnp.zeros((N, D), jnp.float32)
    partials = _sc_propagate(x, srcw, dst, zeros)
    return _tc_finish(partials, W)


# speedup vs baseline: 6.4627x; 1.5474x over previous
import dataclasses
import functools

import jax
import jax.numpy as jnp
from jax import lax
from jax.experimental import pallas as pl
from jax.experimental.pallas import tpu as pltpu
from jax.experimental.pallas import tpu_sc as plsc

N = 10000
E = 320000
D = 128

NUM_CORES = 2
NUM_SUBCORES = 16
NUM_WORKERS = NUM_CORES * NUM_SUBCORES
EDGES_PER_WORKER = E // NUM_WORKERS
CHUNK = 80
K = EDGES_PER_WORKER // CHUNK
KALL = E // CHUNK
ROWS_PER_TILE = 624
TAIL_ROW0 = ROWS_PER_TILE * NUM_SUBCORES
TAIL_ROWS = N - TAIL_ROW0
LANES = 16
SW = 2 * CHUNK


def _sc_propagate(x, srcw, dst, zeros):
    mesh = plsc.VectorSubcoreMesh(core_axis_name="c", subcore_axis_name="s")
    cp = pltpu.CompilerParams()
    if "needs_layout_passes" in pltpu.CompilerParams.__dataclass_fields__:
        cp = dataclasses.replace(cp, needs_layout_passes=False)

    @functools.partial(
        pl.kernel,
        compiler_params=cp,
        out_type=jax.ShapeDtypeStruct((NUM_CORES * N, D), jnp.float32),
        mesh=mesh,
        scratch_types=(
            [pltpu.VMEM((SW,), jnp.int32) for _ in range(4)]
            + [pltpu.VMEM((CHUNK,), jnp.int32) for _ in range(4)]
            + [pltpu.VMEM((CHUNK, D), jnp.float32) for _ in range(2)]
            + [pltpu.VMEM_SHARED((N, D), jnp.float32)]
            + [pltpu.SemaphoreType.DMA for _ in range(8)]
        ),
    )
    def k(x_hbm, sw_hbm, dst_hbm, z_hbm, out_hbm,
          s0, s1, s2, s3, d0, d1, d2, d3, r0, r1, acc,
          se0, se1, se2, se3, sg0, sg1, ss0, ss1):
        sbuf = [s0, s1, s2, s3]
        dbuf = [d0, d1, d2, d3]
        rows = [r0, r1]
        se = [se0, se1, se2, se3]
        sg = [sg0, sg1]
        ss = [ss0, ss1]

        cid = lax.axis_index("c")
        sid = lax.axis_index("s")
        wid = cid * NUM_SUBCORES + sid

        def emit_e(j, eb):
            gw = wid * K + j
            pltpu.async_copy(sw_hbm.at[pl.ds(gw * SW, SW)], sbuf[eb], se[eb])
            pltpu.async_copy(dst_hbm.at[pl.ds(wid * EDGES_PER_WORKER + j * CHUNK,
                                              CHUNK)], dbuf[eb], se[eb])

        def wait_e(eb):
            pltpu.make_async_copy(sw_hbm.at[pl.ds(0, SW)], sbuf[eb],
                                  se[eb]).wait()
            pltpu.make_async_copy(dst_hbm.at[pl.ds(0, CHUNK)], dbuf[eb],
                                  se[eb]).wait()

        def emit_g(eb, rb):
            pltpu.async_copy(x_hbm.at[sbuf[eb].at[pl.ds(0, CHUNK)]],
                             rows[rb], sg[rb])

        def wait_g(rb):
            pltpu.make_async_copy(x_hbm.at[pl.ds(0, CHUNK)], rows[rb],
                                  sg[rb]).wait()

        def emit_s(eb, rb):
            pltpu.async_copy(rows[rb], acc.at[dbuf[eb]], ss[rb], add=True)

        def wait_s(rb):
            pltpu.make_async_copy(rows[rb], acc.at[pl.ds(0, CHUNK)],
                                  ss[rb]).wait()

        def compute(eb, rb):
            for g in range(CHUNK // LANES):
                w16 = plsc.bitcast(sbuf[eb][pl.ds(CHUNK + g * LANES, LANES)],
                                   jnp.float32)
                for el in range(LANES):
                    wb = lax.gather(
                        w16, jnp.full((LANES, 1), el, jnp.int32),
                        lax.GatherDimensionNumbers(
                            offset_dims=(), collapsed_slice_dims=(0,),
                            start_index_map=(0,)),
                        slice_sizes=(1,),
                        mode=lax.GatherScatterMode.PROMISE_IN_BOUNDS)
                    e = g * LANES + el
                    for j in range(D // LANES):
                        sl = pl.ds(j * LANES, LANES)
                        rows[rb][e, sl] = rows[rb][e, sl] * wb

        row0 = sid * ROWS_PER_TILE
        pltpu.sync_copy(z_hbm.at[pl.ds(row0, ROWS_PER_TILE)],
                        acc.at[pl.ds(row0, ROWS_PER_TILE)])

        @pl.when(sid == NUM_SUBCORES - 1)
        def _():
            pltpu.sync_copy(z_hbm.at[pl.ds(TAIL_ROW0, TAIL_ROWS)],
                            acc.at[pl.ds(TAIL_ROW0, TAIL_ROWS)])

        emit_e(0, 0)
        emit_e(1, 1)
        plsc.subcore_barrier()
        wait_e(0)
        emit_g(0, 0)

        @pl.loop(0, K - 1, step=4)
        def _(c0):
            for b in range(4):
                j = c0 + b
                rb = b % 2
                ob = 1 - rb
                wait_g(rb)
                compute(b, rb)
                emit_s(b, rb)

                @pl.when(j + 2 < K)
                def _():
                    emit_e(j + 2, (b + 2) % 4)

                if b == 0:
                    @pl.when(j >= 1)
                    def _():
                        wait_s(ob)
                else:
                    wait_s(ob)
                wait_e((b + 1) % 4)
                emit_g((b + 1) % 4, ob)

        wait_g(0)
        compute(0, 0)
        emit_s(0, 0)
        wait_s(1)
        wait_s(0)

        plsc.subcore_barrier()
        out_base = cid * N + sid * ROWS_PER_TILE
        pltpu.sync_copy(acc.at[pl.ds(row0, ROWS_PER_TILE)],
                        out_hbm.at[pl.ds(out_base, ROWS_PER_TILE)])

        @pl.when(sid == NUM_SUBCORES - 1)
        def _():
            pltpu.sync_copy(acc.at[pl.ds(TAIL_ROW0, TAIL_ROWS)],
                            out_hbm.at[pl.ds(cid * N + TAIL_ROW0, TAIL_ROWS)])

    return k(x, srcw, dst, zeros)


def _tc_finish_body(p0_ref, p1_ref, w_ref, o_ref):
    s = p0_ref[...] + p1_ref[...]
    o_ref[...] = jnp.dot(s, w_ref[...], preferred_element_type=jnp.float32)


def _tc_finish(partials, W):
    blk = 1000
    grid = (N // blk,)
    return pl.pallas_call(
        _tc_finish_body,
        grid=grid,
        in_specs=[
            pl.BlockSpec((blk, D), lambda i: (i, 0)),
            pl.BlockSpec((blk, D), lambda i: (i + N // blk, 0)),
            pl.BlockSpec((D, D), lambda i: (0, 0)),
        ],
        out_specs=pl.BlockSpec((blk, D), lambda i: (i, 0)),
        out_shape=jax.ShapeDtypeStruct((N, D), jnp.float32),
    )(partials, partials, W)


def kernel(x, edge_index, edge_weight, W):
    dst = edge_index[0]
    src = edge_index[1]
    srcw = jnp.concatenate(
        [src.reshape(KALL, CHUNK),
         jax.lax.bitcast_convert_type(edge_weight, jnp.int32).reshape(
             KALL, CHUNK)],
        axis=1).reshape(KALL * SW)
    zeros = jnp.zeros((N, D), jnp.float32)
    partials = _sc_propagate(x, srcw, dst, zeros)
    return _tc_finish(partials, W)


# speedup vs baseline: 7.9604x; 1.2317x over previous
import dataclasses
import functools

import jax
import jax.numpy as jnp
from jax import lax
from jax.experimental import pallas as pl
from jax.experimental.pallas import tpu as pltpu
from jax.experimental.pallas import tpu_sc as plsc

N = 10000
E = 320000
D = 128

NUM_CORES = 2
NUM_SUBCORES = 16
NUM_WORKERS = NUM_CORES * NUM_SUBCORES
EDGES_PER_WORKER = E // NUM_WORKERS
CHUNK = 80
K = EDGES_PER_WORKER // CHUNK
KALL = E // CHUNK
ROWS_PER_TILE = 624
TAIL_ROW0 = ROWS_PER_TILE * NUM_SUBCORES
TAIL_ROWS = N - TAIL_ROW0
LANES = 16
SW = 2 * CHUNK


def _sc_propagate(x, srcw, dst, zeros):
    mesh = plsc.VectorSubcoreMesh(core_axis_name="c", subcore_axis_name="s")
    cp = pltpu.CompilerParams()
    if "needs_layout_passes" in pltpu.CompilerParams.__dataclass_fields__:
        cp = dataclasses.replace(cp, needs_layout_passes=False)

    @functools.partial(
        pl.kernel,
        compiler_params=cp,
        out_type=jax.ShapeDtypeStruct((NUM_CORES * N, D), jnp.float32),
        mesh=mesh,
        scratch_types=(
            [pltpu.VMEM((SW,), jnp.int32) for _ in range(4)]
            + [pltpu.VMEM((CHUNK,), jnp.int32) for _ in range(4)]
            + [pltpu.VMEM((CHUNK, D), jnp.float32) for _ in range(2)]
            + [pltpu.VMEM_SHARED((N, D), jnp.float32)]
            + [pltpu.SemaphoreType.DMA for _ in range(8)]
        ),
    )
    def k(x_hbm, sw_hbm, dst_hbm, z_hbm, out_hbm,
          s0, s1, s2, s3, d0, d1, d2, d3, r0, r1, acc,
          se0, se1, se2, se3, sg0, sg1, ss0, ss1):
        sbuf = [s0, s1, s2, s3]
        dbuf = [d0, d1, d2, d3]
        rows = [r0, r1]
        se = [se0, se1, se2, se3]
        sg = [sg0, sg1]
        ss = [ss0, ss1]

        cid = lax.axis_index("c")
        sid = lax.axis_index("s")
        wid = cid * NUM_SUBCORES + sid

        def emit_e(j, eb):
            gw = wid * K + j
            pltpu.async_copy(sw_hbm.at[pl.ds(gw * SW, SW)], sbuf[eb], se[eb])
            pltpu.async_copy(dst_hbm.at[pl.ds(wid * EDGES_PER_WORKER + j * CHUNK,
                                              CHUNK)], dbuf[eb], se[eb])

        def wait_e(eb):
            pltpu.make_async_copy(sw_hbm.at[pl.ds(0, SW)], sbuf[eb],
                                  se[eb]).wait()
            pltpu.make_async_copy(dst_hbm.at[pl.ds(0, CHUNK)], dbuf[eb],
                                  se[eb]).wait()

        def emit_g(eb, rb):
            pltpu.async_copy(x_hbm.at[sbuf[eb].at[pl.ds(0, CHUNK)]],
                             rows[rb], sg[rb])

        def wait_g(rb):
            pltpu.make_async_copy(x_hbm.at[pl.ds(0, CHUNK)], rows[rb],
                                  sg[rb]).wait()

        def emit_s(eb, rb):
            pltpu.async_copy(rows[rb], acc.at[dbuf[eb]], ss[rb], add=True)

        def wait_s(rb):
            pltpu.make_async_copy(rows[rb], acc.at[pl.ds(0, CHUNK)],
                                  ss[rb]).wait()

        def compute(eb, rb):
            for g in range(CHUNK // LANES):
                w16 = plsc.bitcast(sbuf[eb][pl.ds(CHUNK + g * LANES, LANES)],
                                   jnp.float32)
                for el in range(LANES):
                    wb = lax.gather(
                        w16, jnp.full((LANES, 1), el, jnp.int32),
                        lax.GatherDimensionNumbers(
                            offset_dims=(), collapsed_slice_dims=(0,),
                            start_index_map=(0,)),
                        slice_sizes=(1,),
                        mode=lax.GatherScatterMode.PROMISE_IN_BOUNDS)
                    e = g * LANES + el
                    for j in range(D // LANES):
                        sl = pl.ds(j * LANES, LANES)
                        rows[rb][e, sl] = rows[rb][e, sl] * wb

        row0 = sid * ROWS_PER_TILE
        pltpu.sync_copy(z_hbm.at[pl.ds(row0, ROWS_PER_TILE)],
                        acc.at[pl.ds(row0, ROWS_PER_TILE)])

        @pl.when(sid == NUM_SUBCORES - 1)
        def _():
            pltpu.sync_copy(z_hbm.at[pl.ds(TAIL_ROW0, TAIL_ROWS)],
                            acc.at[pl.ds(TAIL_ROW0, TAIL_ROWS)])

        emit_e(0, 0)
        emit_e(1, 1)
        plsc.subcore_barrier()
        wait_e(0)
        emit_g(0, 0)

        @pl.loop(0, K - 1, step=4)
        def _(c0):
            for b in range(4):
                j = c0 + b
                rb = b % 2
                ob = 1 - rb
                wait_g(rb)
                if b == 0:
                    @pl.when(j >= 1)
                    def _():
                        wait_s(ob)
                else:
                    wait_s(ob)
                wait_e((b + 1) % 4)
                emit_g((b + 1) % 4, ob)

                @pl.when(j + 2 < K)
                def _():
                    emit_e(j + 2, (b + 2) % 4)

                compute(b, rb)
                emit_s(b, rb)

        wait_g(0)
        compute(0, 0)
        emit_s(0, 0)
        wait_s(1)
        wait_s(0)

        plsc.subcore_barrier()
        out_base = cid * N + sid * ROWS_PER_TILE
        pltpu.sync_copy(acc.at[pl.ds(row0, ROWS_PER_TILE)],
                        out_hbm.at[pl.ds(out_base, ROWS_PER_TILE)])

        @pl.when(sid == NUM_SUBCORES - 1)
        def _():
            pltpu.sync_copy(acc.at[pl.ds(TAIL_ROW0, TAIL_ROWS)],
                            out_hbm.at[pl.ds(cid * N + TAIL_ROW0, TAIL_ROWS)])

    return k(x, srcw, dst, zeros)


def _tc_finish_body(p0_ref, p1_ref, w_ref, o_ref):
    s = p0_ref[...] + p1_ref[...]
    o_ref[...] = jnp.dot(s, w_ref[...], preferred_element_type=jnp.float32)


def _tc_finish(partials, W):
    blk = 1000
    grid = (N // blk,)
    return pl.pallas_call(
        _tc_finish_body,
        grid=grid,
        in_specs=[
            pl.BlockSpec((blk, D), lambda i: (i, 0)),
            pl.BlockSpec((blk, D), lambda i: (i + N // blk, 0)),
            pl.BlockSpec((D, D), lambda i: (0, 0)),
        ],
        out_specs=pl.BlockSpec((blk, D), lambda i: (i, 0)),
        out_shape=jax.ShapeDtypeStruct((N, D), jnp.float32),
    )(partials, partials, W)


def kernel(x, edge_index, edge_weight, W):
    dst = edge_index[0]
    src = edge_index[1]
    srcw = jnp.concatenate(
        [src.reshape(KALL, CHUNK),
         jax.lax.bitcast_convert_type(edge_weight, jnp.int32).reshape(
             KALL, CHUNK)],
        axis=1).reshape(KALL * SW)
    zeros = jnp.zeros((N, D), jnp.float32)
    partials = _sc_propagate(x, srcw, dst, zeros)
    return _tc_finish(partials, W)
